# Initial kernel scaffold; baseline (speedup 1.0000x reference)
import jax
import jax.numpy as jnp
from jax.experimental import pallas as pl


def kernel(x, table):
    raise NotImplementedError("write your pallas kernel here")


# speedup vs baseline: 1.0227x; 1.0227x over previous
import functools

import jax
import jax.numpy as jnp
from jax import lax
from jax.experimental import pallas as pl
from jax.experimental.pallas import tpu as pltpu
from jax.experimental.pallas import tpu_sc as plsc

EMBED = 32
B_TOT = 16384 * 50

_NC, _NS = 2, 16
_NW = _NC * _NS
_BPW = B_TOT // _NW
_C = 128
_NCH = _BPW // _C


def _emb_body(x_hbm, table_hbm, out_hbm, idx_v, buf_v, sem):
    wid = lax.axis_index("s") * _NC + lax.axis_index("c")
    base = wid * _BPW
    pltpu.sync_copy(x_hbm.at[pl.ds(base, _BPW)], idx_v)

    def body(j, carry):
        off = j * _C
        pltpu.async_copy(
            table_hbm.at[idx_v.at[pl.ds(off, _C)]], buf_v, sem
        ).wait()
        pltpu.sync_copy(buf_v, out_hbm.at[pl.ds(base + off, _C)])
        return carry

    lax.fori_loop(0, _NCH, body, 0)


def kernel(x, table):
    xf = x.reshape(-1)
    out = pl.kernel(
        _emb_body,
        out_type=jax.ShapeDtypeStruct((B_TOT, EMBED), jnp.float32),
        scratch_types=[
            pltpu.VMEM((_BPW,), jnp.int32),
            pltpu.VMEM((_C, EMBED), jnp.float32),
            pltpu.SemaphoreType.DMA,
        ],
        mesh=plsc.VectorSubcoreMesh(core_axis_name="c", subcore_axis_name="s"),
        compiler_params=pltpu.CompilerParams(use_tc_tiling_on_sc=False),
    )(xf, table)
    return out.reshape(x.shape + (EMBED,))


# speedup vs baseline: 1.1114x; 1.0867x over previous
import jax
import jax.numpy as jnp
from jax import lax
from jax.experimental import pallas as pl
from jax.experimental.pallas import tpu as pltpu
from jax.experimental.pallas import tpu_sc as plsc

EMBED = 32
B_TOT = 16384 * 50

_NC, _NS = 2, 16
_NW = _NC * _NS
_BPW = B_TOT // _NW
_C = 128
_NCH = _BPW // _C
_NBUF = 8
_LA = 4
_NBLK = _NCH // _NBUF


def _emb_body(x_hbm, table_hbm, out_hbm, idx_v, *rest):
    bufs = rest[:_NBUF]
    gsems = rest[_NBUF:2 * _NBUF]
    ssems = rest[2 * _NBUF:3 * _NBUF]
    wid = lax.axis_index("s") * _NC + lax.axis_index("c")
    base = wid * _BPW
    pltpu.sync_copy(x_hbm.at[pl.ds(base, _BPW)], idx_v)

    def g_start(b, j):
        pltpu.async_copy(
            table_hbm.at[idx_v.at[pl.ds(j * _C, _C)]], bufs[b], gsems[b])

    def g_wait(b):
        pltpu.make_async_copy(
            table_hbm.at[idx_v.at[pl.ds(0, _C)]], bufs[b], gsems[b]).wait()

    def s_start(b, j):
        pltpu.async_copy(bufs[b], out_hbm.at[pl.ds(base + j * _C, _C)],
                         ssems[b])

    def s_wait(b):
        pltpu.make_async_copy(bufs[b], out_hbm.at[pl.ds(base, _C)],
                              ssems[b]).wait()

    for j in range(_LA):
        g_start(j, j)

    for b in range(_NBUF):
        jf = b + _LA
        fb = jf % _NBUF
        if jf >= _NBUF:
            s_wait(fb)
        g_start(fb, jf)
        g_wait(b)
        s_start(b, b)

    def block(outer, carry):
        for b in range(_NBUF):
            j = outer * _NBUF + b
            fb = (b + _LA) % _NBUF
            s_wait(fb)
            g_start(fb, j + _LA)
            g_wait(b)
            s_start(b, j)
        return carry

    lax.fori_loop(1, _NBLK - 1, block, 0)

    j0 = (_NBLK - 1) * _NBUF
    for b in range(_NBUF):
        j = j0 + b
        if b < _NBUF - _LA:
            fb = (b + _LA) % _NBUF
            s_wait(fb)
            g_start(fb, j + _LA)
        g_wait(b)
        s_start(b, j)

    for b in range(_NBUF):
        s_wait(b)


def kernel(x, table):
    xf = x.reshape(-1)
    scratch = [pltpu.VMEM((_BPW,), jnp.int32)]
    scratch += [pltpu.VMEM((_C, EMBED), jnp.float32) for _ in range(_NBUF)]
    scratch += [pltpu.SemaphoreType.DMA for _ in range(2 * _NBUF)]
    out = pl.kernel(
        _emb_body,
        out_type=jax.ShapeDtypeStruct((B_TOT, EMBED), jnp.float32),
        scratch_types=scratch,
        mesh=plsc.VectorSubcoreMesh(core_axis_name="c", subcore_axis_name="s"),
        compiler_params=pltpu.CompilerParams(use_tc_tiling_on_sc=False),
    )(xf, table)
    return out.reshape(x.shape + (EMBED,))


# speedup vs baseline: 1.8053x; 1.6243x over previous
import jax
import jax.numpy as jnp
from jax import lax
from jax.experimental import pallas as pl
from jax.experimental.pallas import tpu as pltpu
from jax.experimental.pallas import tpu_sc as plsc

EMBED = 32
BATCH = 16384
HIST = 50

_NC, _NS = 2, 16
_NW = _NC * _NS
_RPW = BATCH // _NW
_RPC = 4
_NCH = _RPW // _RPC
_NBUF = 8
_LA = 4
_NBLK = _NCH // _NBUF


def _emb_body(x_hbm, table_hbm, out_hbm, idx_v, *rest):
    bufs = rest[:_NBUF]
    gsems = rest[_NBUF:2 * _NBUF]
    ssems = rest[2 * _NBUF:3 * _NBUF]
    wid = lax.axis_index("s") * _NC + lax.axis_index("c")
    base = wid * _RPW
    pltpu.sync_copy(x_hbm.at[pl.ds(base, _RPW)], idx_v)

    def g_start(b, c):
        for rr in range(_RPC):
            pltpu.async_copy(
                table_hbm.at[idx_v.at[c * _RPC + rr]],
                bufs[b].at[rr], gsems[b])

    def g_wait(b):
        pltpu.make_async_copy(
            out_hbm.at[pl.ds(0, _RPC)], bufs[b], gsems[b]).wait()

    def s_start(b, c):
        pltpu.async_copy(
            bufs[b], out_hbm.at[pl.ds(base + c * _RPC, _RPC)], ssems[b])

    def s_wait(b):
        pltpu.make_async_copy(
            bufs[b], out_hbm.at[pl.ds(0, _RPC)], ssems[b]).wait()

    for c in range(_LA):
        g_start(c, c)

    for b in range(_NBUF):
        cf = b + _LA
        fb = cf % _NBUF
        if cf >= _NBUF:
            s_wait(fb)
        g_start(fb, cf)
        g_wait(b)
        s_start(b, b)

    def block(outer, carry):
        for b in range(_NBUF):
            c = outer * _NBUF + b
            fb = (b + _LA) % _NBUF
            s_wait(fb)
            g_start(fb, c + _LA)
            g_wait(b)
            s_start(b, c)
        return carry

    lax.fori_loop(1, _NBLK - 1, block, 0)

    c0 = (_NBLK - 1) * _NBUF
    for b in range(_NBUF):
        c = c0 + b
        if b < _NBUF - _LA:
            fb = (b + _LA) % _NBUF
            s_wait(fb)
            g_start(fb, c + _LA)
        g_wait(b)
        s_start(b, c)

    for b in range(_NBUF):
        s_wait(b)


def kernel(x, table):
    scratch = [pltpu.VMEM((_RPW, HIST), jnp.int32)]
    scratch += [pltpu.VMEM((_RPC, HIST, EMBED), jnp.float32)
                for _ in range(_NBUF)]
    scratch += [pltpu.SemaphoreType.DMA for _ in range(2 * _NBUF)]
    return pl.kernel(
        _emb_body,
        out_type=jax.ShapeDtypeStruct((BATCH, HIST, EMBED), jnp.float32),
        scratch_types=scratch,
        mesh=plsc.VectorSubcoreMesh(core_axis_name="c", subcore_axis_name="s"),
        compiler_params=pltpu.CompilerParams(use_tc_tiling_on_sc=False),
    )(x, table)
